# baseline (speedup 1.0000x reference)
import functools

import jax
import jax.numpy as jnp
from jax import lax
from jax.experimental import pallas as pl
from jax.experimental.pallas import tpu as pltpu
from jax.experimental.pallas import tpu_sc as plsc

N = 10000
E = 320000
D = 64
NC = 2
NS = 16
NW = NC * NS
CH = 128
CPW = 80
CPA = 160
DH = D // NC
NB = 4
NG = CPA // NB
E_PAD = NW * CPW * CH
NACC = 10240
NPAD = NACC

_mesh = plsc.VectorSubcoreMesh(core_axis_name="c", subcore_axis_name="s")
_sc_params = pltpu.CompilerParams(use_tc_tiling_on_sc=False)


@functools.partial(
    pl.kernel,
    mesh=_mesh,
    compiler_params=_sc_params,
    out_type=(
        jax.ShapeDtypeStruct((NW * CPW, CH), jnp.int32),
        jax.ShapeDtypeStruct((NC, NACC, 16), jnp.float32),
    ),
    scratch_types=[
        pltpu.VMEM((CPW, CH), jnp.int32),
        pltpu.VMEM((CPW, CH), jnp.int32),
        pltpu.VMEM((CPW, CH), jnp.int32),
        pltpu.VMEM((CH, 16), jnp.float32),
        pltpu.VMEM_SHARED((NACC, 16), jnp.float32),
        pltpu.SemaphoreType.DMA,
    ],
)
def _sc_prep(src_h, dst_h, ones_h, z16_h, se_h, degp_h, src_v, dst_v, se_v,
             ones_v, acc, sem):
    c = lax.axis_index("c")
    s = lax.axis_index("s")
    wid = c * NS + s
    rows = NACC // NS
    pltpu.sync_copy(ones_h, ones_v)
    pltpu.sync_copy(src_h.at[pl.ds(wid * CPW, CPW)], src_v)
    pltpu.sync_copy(dst_h.at[pl.ds(wid * CPW, CPW)], dst_v)
    pltpu.sync_copy(z16_h.at[pl.ds(s * rows, rows)], acc.at[pl.ds(s * rows, rows)])
    plsc.subcore_barrier()

    def chunk(j, carry):
        def vec(i, carry2):
            s16 = src_v[j, pl.ds(i * 16, 16)]
            d16 = dst_v[j, pl.ds(i * 16, 16)]
            se_v[j, pl.ds(i * 16, 16)] = jnp.where(s16 != d16, s16, N)
            return carry2

        lax.fori_loop(0, CH // 16, vec, 0)
        pltpu.async_copy(ones_v, acc.at[se_v.at[j]], sem, add=True)
        return carry

    lax.fori_loop(0, CPW, chunk, 0)

    def drain(j, carry):
        pltpu.make_async_copy(ones_v, acc.at[se_v.at[0]], sem).wait()
        return carry

    lax.fori_loop(0, CPW, drain, 0)
    pltpu.sync_copy(se_v, se_h.at[pl.ds(wid * CPW, CPW)])
    plsc.subcore_barrier()
    pltpu.sync_copy(acc.at[pl.ds(s * rows, rows)], degp_h.at[c, pl.ds(s * rows, rows)])


@functools.partial(
    pl.kernel,
    mesh=_mesh,
    compiler_params=_sc_params,
    out_type=jax.ShapeDtypeStruct((NC, NACC, DH), jnp.float32),
    scratch_types=[
        pltpu.VMEM((CPA, CH), jnp.int32),
        pltpu.VMEM((CPA, CH), jnp.int32),
        pltpu.VMEM((NB, CH, DH), jnp.float32),
        pltpu.VMEM_SHARED((NACC, DH), jnp.float32),
        pltpu.VMEM_SHARED((NACC, DH), jnp.float32),
    ]
    + [pltpu.SemaphoreType.DMA] * (2 * NB),
)
def _sc_agg(tab_h, se_h, dst_h, z_h, aggp_h, sidx_v, didx_v, rows_v, acc,
            tab_v, *sems):
    gsem = sems[:NB]
    ssem = sems[NB:]
    c = lax.axis_index("c")
    s = lax.axis_index("s")
    rows = NACC // NS
    pltpu.sync_copy(z_h, acc.at[pl.ds(s * rows, rows)])
    pltpu.sync_copy(tab_h.at[c, pl.ds(s * rows, rows)],
                    tab_v.at[pl.ds(s * rows, rows)])
    pltpu.sync_copy(se_h.at[pl.ds(s * CPA, CPA)], sidx_v)
    pltpu.sync_copy(dst_h.at[pl.ds(s * CPA, CPA)], didx_v)
    plsc.subcore_barrier()

    def wait_gather(b):
        pltpu.make_async_copy(tab_v.at[sidx_v.at[0]], rows_v.at[b], gsem[b]).wait()

    def wait_scatter(b):
        pltpu.make_async_copy(rows_v.at[b], acc.at[didx_v.at[0]], ssem[b]).wait()

    for b in range(NB):
        pltpu.async_copy(tab_v.at[sidx_v.at[b]], rows_v.at[b], gsem[b])

    def group(g, carry):
        for b in range(NB):
            j = g * NB + b
            wait_gather(b)
            pltpu.async_copy(rows_v.at[b], acc.at[didx_v.at[j]], ssem[b], add=True)
        for b in range(NB):
            j2 = (g + 1) * NB + b
            wait_scatter(b)
            pltpu.async_copy(tab_v.at[sidx_v.at[j2]], rows_v.at[b], gsem[b])
        return carry

    lax.fori_loop(0, NG - 1, group, 0)
    for b in range(NB):
        j = (NG - 1) * NB + b
        wait_gather(b)
        pltpu.async_copy(rows_v.at[b], acc.at[didx_v.at[j]], ssem[b], add=True)
    for b in range(NB):
        wait_scatter(b)
    plsc.subcore_barrier()
    pltpu.sync_copy(acc.at[pl.ds(s * rows, rows)], aggp_h.at[c, pl.ds(s * rows, rows)])


def _tc0_body(x, w10, w11, xw0_o, xw1_o):
    xw0_o[...] = jnp.dot(x[...], w10[...], preferred_element_type=jnp.float32)
    xw1_o[...] = jnp.dot(x[...], w11[...], preferred_element_type=jnp.float32)


_tc0 = pl.pallas_call(
    _tc0_body,
    out_shape=(
        jax.ShapeDtypeStruct((N, D), jnp.float32),
        jax.ShapeDtypeStruct((N, D), jnp.float32),
    ),
)


def _tc2_body(hpre, w20, w21, hw0_o, hw1_o):
    h = jnp.maximum(hpre[...], 0.0)
    hw0_o[...] = jnp.dot(h, w20[...], preferred_element_type=jnp.float32)
    hw1_o[...] = jnp.dot(h, w21[...], preferred_element_type=jnp.float32)


_tc2 = pl.pallas_call(
    _tc2_body,
    out_shape=(
        jax.ShapeDtypeStruct((N, D), jnp.float32),
        jax.ShapeDtypeStruct((N, D), jnp.float32),
    ),
)


def _tc3_body(opre, out_o):
    o = opre[...]
    m = jnp.max(o, axis=1, keepdims=True)
    lse = jnp.log(jnp.sum(jnp.exp(o - m), axis=1, keepdims=True)) + m
    out_o[...] = o - lse


_tc3 = pl.pallas_call(
    _tc3_body,
    out_shape=jax.ShapeDtypeStruct((N, D), jnp.float32),
)


def _split_pad(ys):
    parts = [
        jnp.pad(ys[:, c * DH : (c + 1) * DH], ((0, NACC - N), (0, 0)))
        for c in range(NC)
    ]
    return jnp.stack(parts)


def _cat(aggp):
    return jnp.concatenate([aggp[0, :N, :], aggp[1, :N, :]], axis=1)


_tc3 = pl.pallas_call(
    _tc3_body,
    out_shape=jax.ShapeDtypeStruct((N, D), jnp.float32),
)


def kernel(x, edge_index, W1_0, W1_1, b1, W2_0, W2_1, b2):
    src = edge_index[0].astype(jnp.int32)
    dst = edge_index[1].astype(jnp.int32)
    pad = E_PAD - E
    srcp = jnp.concatenate([src, jnp.zeros((pad,), jnp.int32)]).reshape(-1, CH)
    dstp = jnp.concatenate([dst, jnp.zeros((pad,), jnp.int32)]).reshape(-1, CH)
    ones_rows = jnp.ones((CH, 16), jnp.float32)
    z16 = jnp.zeros((NACC, 16), jnp.float32)
    z64 = jnp.zeros((NACC // NS, DH), jnp.float32)

    se, degp = _sc_prep(srcp, dstp, ones_rows, z16)
    xw0, xw1 = _tc0(x, W1_0, W1_1)
    deg = degp[0, :N, 0:1] + degp[1, :N, 0:1]
    dis = jnp.where(deg > 0, lax.rsqrt(jnp.maximum(deg, 1e-12)), 0.0)
    agg1 = _sc_agg(_split_pad(dis * xw1), se, dstp, z64)
    hpre = xw0 - dis * _cat(agg1) + b1.reshape(1, D)
    hw0, hw1 = _tc2(hpre, W2_0, W2_1)
    agg2 = _sc_agg(_split_pad(dis * hw1), se, dstp, z64)
    return _tc3(hw0 - dis * _cat(agg2) + b2.reshape(1, D))


# speedup vs baseline: 1.1023x; 1.1023x over previous
import functools

import jax
import jax.numpy as jnp
from jax import lax
from jax.experimental import pallas as pl
from jax.experimental.pallas import tpu as pltpu
from jax.experimental.pallas import tpu_sc as plsc

N = 10000
E = 320000
D = 64
NC = 2
NS = 16
NW = NC * NS
CH = 128
CPW = 80
CPA = 80
NB = 2
NG = CPA // NB
E_PAD = NW * CPW * CH
NACC = 10240
NPAD = NACC

_mesh = plsc.VectorSubcoreMesh(core_axis_name="c", subcore_axis_name="s")
_sc_params = pltpu.CompilerParams(use_tc_tiling_on_sc=False)


@functools.partial(
    pl.kernel,
    mesh=_mesh,
    compiler_params=_sc_params,
    out_type=(
        jax.ShapeDtypeStruct((NW * CPW, CH), jnp.int32),
        jax.ShapeDtypeStruct((NC, NACC, 16), jnp.float32),
    ),
    scratch_types=[
        pltpu.VMEM((CPW, CH), jnp.int32),
        pltpu.VMEM((CPW, CH), jnp.int32),
        pltpu.VMEM((CPW, CH), jnp.int32),
        pltpu.VMEM((CH, 16), jnp.float32),
        pltpu.VMEM_SHARED((NACC, 16), jnp.float32),
        pltpu.SemaphoreType.DMA,
    ],
)
def _sc_prep(src_h, dst_h, ones_h, z16_h, se_h, degp_h, src_v, dst_v, se_v,
             ones_v, acc, sem):
    c = lax.axis_index("c")
    s = lax.axis_index("s")
    wid = c * NS + s
    rows = NACC // NS
    pltpu.sync_copy(ones_h, ones_v)
    pltpu.sync_copy(src_h.at[pl.ds(wid * CPW, CPW)], src_v)
    pltpu.sync_copy(dst_h.at[pl.ds(wid * CPW, CPW)], dst_v)
    pltpu.sync_copy(z16_h.at[pl.ds(s * rows, rows)], acc.at[pl.ds(s * rows, rows)])
    plsc.subcore_barrier()

    def chunk(j, carry):
        def vec(i, carry2):
            s16 = src_v[j, pl.ds(i * 16, 16)]
            d16 = dst_v[j, pl.ds(i * 16, 16)]
            se_v[j, pl.ds(i * 16, 16)] = jnp.where(s16 != d16, s16, N)
            return carry2

        lax.fori_loop(0, CH // 16, vec, 0)
        pltpu.async_copy(ones_v, acc.at[se_v.at[j]], sem, add=True)
        return carry

    lax.fori_loop(0, CPW, chunk, 0)

    def drain(j, carry):
        pltpu.make_async_copy(ones_v, acc.at[se_v.at[0]], sem).wait()
        return carry

    lax.fori_loop(0, CPW, drain, 0)
    pltpu.sync_copy(se_v, se_h.at[pl.ds(wid * CPW, CPW)])
    plsc.subcore_barrier()
    pltpu.sync_copy(acc.at[pl.ds(s * rows, rows)], degp_h.at[c, pl.ds(s * rows, rows)])


@functools.partial(
    pl.kernel,
    mesh=_mesh,
    compiler_params=_sc_params,
    out_type=jax.ShapeDtypeStruct((NC, NACC, D), jnp.float32),
    scratch_types=[
        pltpu.VMEM((CPA, CH), jnp.int32),
        pltpu.VMEM((CPA, CH), jnp.int32),
        pltpu.VMEM((NB, CH, D), jnp.float32),
        pltpu.VMEM_SHARED((NACC, D), jnp.float32),
        pltpu.VMEM_SHARED((NACC, D), jnp.float32),
    ]
    + [pltpu.SemaphoreType.DMA] * (2 * NB),
)
def _sc_agg(tab_h, se_h, dst_h, z_h, aggp_h, sidx_v, didx_v, rows_v, acc,
            tab_v, *sems):
    gsem = sems[:NB]
    ssem = sems[NB:]
    c = lax.axis_index("c")
    s = lax.axis_index("s")
    wid = c * NS + s
    rows = NACC // NS
    pltpu.sync_copy(z_h, acc.at[pl.ds(s * rows, rows)])
    pltpu.sync_copy(tab_h.at[pl.ds(s * rows, rows)],
                    tab_v.at[pl.ds(s * rows, rows)])
    pltpu.sync_copy(se_h.at[pl.ds(wid * CPA, CPA)], sidx_v)
    pltpu.sync_copy(dst_h.at[pl.ds(wid * CPA, CPA)], didx_v)
    plsc.subcore_barrier()

    def wait_gather(b):
        pltpu.make_async_copy(tab_v.at[sidx_v.at[0]], rows_v.at[b], gsem[b]).wait()

    def wait_scatter(b):
        pltpu.make_async_copy(rows_v.at[b], acc.at[didx_v.at[0]], ssem[b]).wait()

    for b in range(NB):
        pltpu.async_copy(tab_v.at[sidx_v.at[b]], rows_v.at[b], gsem[b])

    def group(g, carry):
        for b in range(NB):
            j = g * NB + b
            wait_gather(b)
            pltpu.async_copy(rows_v.at[b], acc.at[didx_v.at[j]], ssem[b], add=True)
        for b in range(NB):
            j2 = (g + 1) * NB + b
            wait_scatter(b)
            pltpu.async_copy(tab_v.at[sidx_v.at[j2]], rows_v.at[b], gsem[b])
        return carry

    lax.fori_loop(0, NG - 1, group, 0)
    for b in range(NB):
        j = (NG - 1) * NB + b
        wait_gather(b)
        pltpu.async_copy(rows_v.at[b], acc.at[didx_v.at[j]], ssem[b], add=True)
    for b in range(NB):
        wait_scatter(b)
    plsc.subcore_barrier()
    pltpu.sync_copy(acc.at[pl.ds(s * rows, rows)], aggp_h.at[c, pl.ds(s * rows, rows)])


def _tc0_body(x, w10, w11, xw0_o, xw1_o):
    xw0_o[...] = jnp.dot(x[...], w10[...], preferred_element_type=jnp.float32)
    xw1_o[...] = jnp.dot(x[...], w11[...], preferred_element_type=jnp.float32)


_tc0 = pl.pallas_call(
    _tc0_body,
    out_shape=(
        jax.ShapeDtypeStruct((N, D), jnp.float32),
        jax.ShapeDtypeStruct((N, D), jnp.float32),
    ),
)


def _tc2_body(hpre, w20, w21, hw0_o, hw1_o):
    h = jnp.maximum(hpre[...], 0.0)
    hw0_o[...] = jnp.dot(h, w20[...], preferred_element_type=jnp.float32)
    hw1_o[...] = jnp.dot(h, w21[...], preferred_element_type=jnp.float32)


_tc2 = pl.pallas_call(
    _tc2_body,
    out_shape=(
        jax.ShapeDtypeStruct((N, D), jnp.float32),
        jax.ShapeDtypeStruct((N, D), jnp.float32),
    ),
)


def _tc3_body(opre, out_o):
    o = opre[...]
    m = jnp.max(o, axis=1, keepdims=True)
    lse = jnp.log(jnp.sum(jnp.exp(o - m), axis=1, keepdims=True)) + m
    out_o[...] = o - lse


_tc3 = pl.pallas_call(
    _tc3_body,
    out_shape=jax.ShapeDtypeStruct((N, D), jnp.float32),
)


def _pad_rows(ys):
    return jnp.pad(ys, ((0, NACC - N), (0, 0)))


def _psum(aggp):
    return aggp[0, :N, :] + aggp[1, :N, :]


def kernel(x, edge_index, W1_0, W1_1, b1, W2_0, W2_1, b2):
    src = edge_index[0].astype(jnp.int32)
    dst = edge_index[1].astype(jnp.int32)
    pad = E_PAD - E
    srcp = jnp.concatenate([src, jnp.zeros((pad,), jnp.int32)]).reshape(-1, CH)
    dstp = jnp.concatenate([dst, jnp.zeros((pad,), jnp.int32)]).reshape(-1, CH)
    ones_rows = jnp.ones((CH, 16), jnp.float32)
    z16 = jnp.zeros((NACC, 16), jnp.float32)
    z64 = jnp.zeros((NACC // NS, D), jnp.float32)

    se, degp = _sc_prep(srcp, dstp, ones_rows, z16)
    xw0, xw1 = _tc0(x, W1_0, W1_1)
    deg = degp[0, :N, 0:1] + degp[1, :N, 0:1]
    dis = jnp.where(deg > 0, lax.rsqrt(jnp.maximum(deg, 1e-12)), 0.0)
    agg1 = _sc_agg(_pad_rows(dis * xw1), se, dstp, z64)
    hpre = xw0 - dis * _psum(agg1) + b1.reshape(1, D)
    hw0, hw1 = _tc2(hpre, W2_0, W2_1)
    agg2 = _sc_agg(_pad_rows(dis * hw1), se, dstp, z64)
    return _tc3(hw0 - dis * _psum(agg2) + b2.reshape(1, D))


# speedup vs baseline: 1.1166x; 1.0129x over previous
import functools

import jax
import jax.numpy as jnp
from jax import lax
from jax.experimental import pallas as pl
from jax.experimental.pallas import tpu as pltpu
from jax.experimental.pallas import tpu_sc as plsc

N = 10000
E = 320000
D = 64
NC = 2
NS = 16
NW = NC * NS
CH = 128
CPW = 80
CPA = 80
NB = 2
NG = CPA // NB
E_PAD = NW * CPW * CH
NACC = 10240
NPAD = NACC

_mesh = plsc.VectorSubcoreMesh(core_axis_name="c", subcore_axis_name="s")
_sc_params = pltpu.CompilerParams(use_tc_tiling_on_sc=False)


@functools.partial(
    pl.kernel,
    mesh=_mesh,
    compiler_params=_sc_params,
    out_type=(
        jax.ShapeDtypeStruct((NW * CPW, CH), jnp.int32),
        jax.ShapeDtypeStruct((NC, NACC, 16), jnp.float32),
    ),
    scratch_types=[
        pltpu.VMEM((CPW, CH), jnp.int32),
        pltpu.VMEM((CPW, CH), jnp.int32),
        pltpu.VMEM((CPW, CH), jnp.int32),
        pltpu.VMEM((CH, 16), jnp.float32),
        pltpu.VMEM_SHARED((NACC, 16), jnp.float32),
        pltpu.SemaphoreType.DMA,
    ],
)
def _sc_prep(src_h, dst_h, ones_h, z16_h, se_h, degp_h, src_v, dst_v, se_v,
             ones_v, acc, sem):
    c = lax.axis_index("c")
    s = lax.axis_index("s")
    wid = c * NS + s
    rows = NACC // NS
    pltpu.async_copy(ones_h, ones_v, sem)
    pltpu.async_copy(src_h.at[pl.ds(wid * CPW, CPW)], src_v, sem)
    pltpu.async_copy(dst_h.at[pl.ds(wid * CPW, CPW)], dst_v, sem)
    pltpu.async_copy(z16_h.at[pl.ds(s * rows, rows)], acc.at[pl.ds(s * rows, rows)], sem)
    pltpu.make_async_copy(ones_h, ones_v, sem).wait()
    pltpu.make_async_copy(src_h.at[pl.ds(wid * CPW, CPW)], src_v, sem).wait()
    pltpu.make_async_copy(dst_h.at[pl.ds(wid * CPW, CPW)], dst_v, sem).wait()
    pltpu.make_async_copy(z16_h.at[pl.ds(s * rows, rows)],
                          acc.at[pl.ds(s * rows, rows)], sem).wait()
    plsc.subcore_barrier()

    def chunk(j, carry):
        def vec(i, carry2):
            s16 = src_v[j, pl.ds(i * 16, 16)]
            d16 = dst_v[j, pl.ds(i * 16, 16)]
            se_v[j, pl.ds(i * 16, 16)] = jnp.where(s16 != d16, s16, N)
            return carry2

        lax.fori_loop(0, CH // 16, vec, 0)
        pltpu.async_copy(ones_v, acc.at[se_v.at[j]], sem, add=True)
        return carry

    lax.fori_loop(0, CPW, chunk, 0)

    def drain(j, carry):
        pltpu.make_async_copy(ones_v, acc.at[se_v.at[0]], sem).wait()
        return carry

    lax.fori_loop(0, CPW, drain, 0)
    pltpu.sync_copy(se_v, se_h.at[pl.ds(wid * CPW, CPW)])
    plsc.subcore_barrier()
    pltpu.sync_copy(acc.at[pl.ds(s * rows, rows)], degp_h.at[c, pl.ds(s * rows, rows)])


@functools.partial(
    pl.kernel,
    mesh=_mesh,
    compiler_params=_sc_params,
    out_type=jax.ShapeDtypeStruct((NC, NACC, D), jnp.float32),
    scratch_types=[
        pltpu.VMEM((CPA, CH), jnp.int32),
        pltpu.VMEM((CPA, CH), jnp.int32),
        pltpu.VMEM((NB, CH, D), jnp.float32),
        pltpu.VMEM_SHARED((NACC, D), jnp.float32),
        pltpu.VMEM_SHARED((NACC, D), jnp.float32),
    ]
    + [pltpu.SemaphoreType.DMA] * (2 * NB + 1),
)
def _sc_agg(tab_h, se_h, dst_h, z_h, aggp_h, sidx_v, didx_v, rows_v, acc,
            tab_v, *sems):
    gsem = sems[:NB]
    ssem = sems[NB : 2 * NB]
    stsem = sems[2 * NB]
    c = lax.axis_index("c")
    s = lax.axis_index("s")
    wid = c * NS + s
    rows = NACC // NS
    pltpu.async_copy(z_h, acc.at[pl.ds(s * rows, rows)], stsem)
    pltpu.async_copy(tab_h.at[pl.ds(s * rows, rows)],
                     tab_v.at[pl.ds(s * rows, rows)], stsem)
    pltpu.async_copy(se_h.at[pl.ds(wid * CPA, CPA)], sidx_v, stsem)
    pltpu.async_copy(dst_h.at[pl.ds(wid * CPA, CPA)], didx_v, stsem)
    pltpu.make_async_copy(z_h, acc.at[pl.ds(s * rows, rows)], stsem).wait()
    pltpu.make_async_copy(tab_h.at[pl.ds(s * rows, rows)],
                          tab_v.at[pl.ds(s * rows, rows)], stsem).wait()
    pltpu.make_async_copy(se_h.at[pl.ds(wid * CPA, CPA)], sidx_v, stsem).wait()
    pltpu.make_async_copy(dst_h.at[pl.ds(wid * CPA, CPA)], didx_v, stsem).wait()
    plsc.subcore_barrier()

    def wait_gather(b):
        pltpu.make_async_copy(tab_v.at[sidx_v.at[0]], rows_v.at[b], gsem[b]).wait()

    def wait_scatter(b):
        pltpu.make_async_copy(rows_v.at[b], acc.at[didx_v.at[0]], ssem[b]).wait()

    for b in range(NB):
        pltpu.async_copy(tab_v.at[sidx_v.at[b]], rows_v.at[b], gsem[b])

    def group(g, carry):
        for b in range(NB):
            j = g * NB + b
            wait_gather(b)
            pltpu.async_copy(rows_v.at[b], acc.at[didx_v.at[j]], ssem[b], add=True)
        for b in range(NB):
            j2 = (g + 1) * NB + b
            wait_scatter(b)
            pltpu.async_copy(tab_v.at[sidx_v.at[j2]], rows_v.at[b], gsem[b])
        return carry

    lax.fori_loop(0, NG - 1, group, 0)
    for b in range(NB):
        j = (NG - 1) * NB + b
        wait_gather(b)
        pltpu.async_copy(rows_v.at[b], acc.at[didx_v.at[j]], ssem[b], add=True)
    for b in range(NB):
        wait_scatter(b)
    plsc.subcore_barrier()
    pltpu.sync_copy(acc.at[pl.ds(s * rows, rows)], aggp_h.at[c, pl.ds(s * rows, rows)])


def _tc0_body(x, w10, w11, xw0_o, xw1_o):
    xw0_o[...] = jnp.dot(x[...], w10[...], preferred_element_type=jnp.float32)
    xw1_o[...] = jnp.dot(x[...], w11[...], preferred_element_type=jnp.float32)


_tc0 = pl.pallas_call(
    _tc0_body,
    out_shape=(
        jax.ShapeDtypeStruct((N, D), jnp.float32),
        jax.ShapeDtypeStruct((N, D), jnp.float32),
    ),
)


def _tc2_body(hpre, w20, w21, hw0_o, hw1_o):
    h = jnp.maximum(hpre[...], 0.0)
    hw0_o[...] = jnp.dot(h, w20[...], preferred_element_type=jnp.float32)
    hw1_o[...] = jnp.dot(h, w21[...], preferred_element_type=jnp.float32)


_tc2 = pl.pallas_call(
    _tc2_body,
    out_shape=(
        jax.ShapeDtypeStruct((N, D), jnp.float32),
        jax.ShapeDtypeStruct((N, D), jnp.float32),
    ),
)


def _tc3_body(opre, out_o):
    o = opre[...]
    m = jnp.max(o, axis=1, keepdims=True)
    lse = jnp.log(jnp.sum(jnp.exp(o - m), axis=1, keepdims=True)) + m
    out_o[...] = o - lse


_tc3 = pl.pallas_call(
    _tc3_body,
    out_shape=jax.ShapeDtypeStruct((N, D), jnp.float32),
)


def _pad_rows(ys):
    return jnp.pad(ys, ((0, NACC - N), (0, 0)))


def _psum(aggp):
    return aggp[0, :N, :] + aggp[1, :N, :]


def kernel(x, edge_index, W1_0, W1_1, b1, W2_0, W2_1, b2):
    src = edge_index[0].astype(jnp.int32)
    dst = edge_index[1].astype(jnp.int32)
    pad = E_PAD - E
    srcp = jnp.concatenate([src, jnp.zeros((pad,), jnp.int32)]).reshape(-1, CH)
    dstp = jnp.concatenate([dst, jnp.zeros((pad,), jnp.int32)]).reshape(-1, CH)
    ones_rows = jnp.ones((CH, 16), jnp.float32)
    z16 = jnp.zeros((NACC, 16), jnp.float32)
    z64 = jnp.zeros((NACC // NS, D), jnp.float32)

    se, degp = _sc_prep(srcp, dstp, ones_rows, z16)
    xw0, xw1 = _tc0(x, W1_0, W1_1)
    deg = degp[0, :N, 0:1] + degp[1, :N, 0:1]
    dis = jnp.where(deg > 0, lax.rsqrt(jnp.maximum(deg, 1e-12)), 0.0)
    agg1 = _sc_agg(_pad_rows(dis * xw1), se, dstp, z64)
    hpre = xw0 - dis * _psum(agg1) + b1.reshape(1, D)
    hw0, hw1 = _tc2(hpre, W2_0, W2_1)
    agg2 = _sc_agg(_pad_rows(dis * hw1), se, dstp, z64)
    return _tc3(hw0 - dis * _psum(agg2) + b2.reshape(1, D))
